# baseline (speedup 1.0000x reference)
import jax
import jax.numpy as jnp
from jax.experimental import pallas as pl
from jax.experimental.pallas import tpu as pltpu

_N_CHUNKS = 32
_CHUNK_B = 2


def _body(pos_ref, kv_ref, cache_ref, out_ref, sems, tok_sem):
    cps = []
    for c in range(_N_CHUNKS):
        sl = pl.ds(c * _CHUNK_B, _CHUNK_B)
        cp = pltpu.make_async_copy(cache_ref.at[sl], out_ref.at[sl], sems.at[c])
        cp.start()
        cps.append(cp)
    for cp in cps:
        cp.wait()
    tok = pltpu.make_async_copy(kv_ref, out_ref.at[:, pl.ds(pos_ref[0], 1), :], tok_sem)
    tok.start()
    tok.wait()


def kernel(kv, start_pos, kv_cache):
    bsz, _, head = kv.shape
    win = kv_cache.shape[1]
    pos = jnp.reshape(jnp.asarray(start_pos, jnp.int32) % win, (1,))
    cache = kv_cache[:bsz]
    out = pl.pallas_call(
        _body,
        out_shape=jax.ShapeDtypeStruct(cache.shape, cache.dtype),
        in_specs=[
            pl.BlockSpec(memory_space=pltpu.SMEM),
            pl.BlockSpec(memory_space=pltpu.VMEM),
            pl.BlockSpec(memory_space=pltpu.HBM),
        ],
        out_specs=pl.BlockSpec(memory_space=pltpu.HBM),
        scratch_shapes=[
            pltpu.SemaphoreType.DMA((_N_CHUNKS,)),
            pltpu.SemaphoreType.DMA,
        ],
    )(pos, kv, cache)
    return out


# speedup vs baseline: 48.9447x; 48.9447x over previous
import jax
import jax.numpy as jnp
from jax.experimental import pallas as pl
from jax.experimental.pallas import tpu as pltpu

_B_BLK = 8
_W_BLK = 2048


def _body(pos_ref, kv_ref, cache_ref, out_ref):
    j = pl.program_id(1)
    local = pos_ref[0] - j * _W_BLK
    out_ref[...] = cache_ref[...]

    @pl.when((local >= 0) & (local < _W_BLK))
    def _():
        out_ref[:, pl.ds(local, 1), :] = kv_ref[...]


def kernel(kv, start_pos, kv_cache):
    bsz, _, head = kv.shape
    win = kv_cache.shape[1]
    pos = jnp.reshape(jnp.asarray(start_pos, jnp.int32) % win, (1,))
    cache = kv_cache[:bsz]
    out = pl.pallas_call(
        _body,
        grid=(bsz // _B_BLK, win // _W_BLK),
        out_shape=jax.ShapeDtypeStruct(cache.shape, cache.dtype),
        in_specs=[
            pl.BlockSpec(memory_space=pltpu.SMEM),
            pl.BlockSpec((_B_BLK, 1, head), lambda i, j: (i, 0, 0)),
            pl.BlockSpec((_B_BLK, _W_BLK, head), lambda i, j: (i, j, 0)),
        ],
        out_specs=pl.BlockSpec((_B_BLK, _W_BLK, head), lambda i, j: (i, j, 0)),
    )(pos, kv, cache)
    return out
